# baseline (speedup 1.0000x reference)
import functools

import jax
import jax.numpy as jnp
from jax import lax
from jax.experimental import pallas as pl
from jax.experimental.pallas import tpu as pltpu
from jax.experimental.pallas import tpu_sc as plsc

_CHUNK = 128
_NSUB = 16
_NBUF = 4


def _sc_accumulate(token2, idx2, dstp, attr, z_a, z_b, z_c):
    n_pad = z_a.shape[0]
    ep = dstp.shape[0]
    e_real = attr.shape[0]
    chunk = _CHUNK
    ept = ep // _NSUB
    n_chunks = ept // chunk
    assert ept % chunk == 0 and n_chunks % _NBUF == 0
    assert n_pad % (8 * _NSUB) == 0
    rpt = n_pad // _NSUB

    mesh = plsc.VectorSubcoreMesh(core_axis_name="c", subcore_axis_name="s")

    @functools.partial(
        pl.kernel,
        compiler_params=pltpu.CompilerParams(use_tc_tiling_on_sc=False,
                                             needs_layout_passes=False),
        out_type=(
            jax.ShapeDtypeStruct((n_pad, 192), jnp.float32),
            jax.ShapeDtypeStruct((n_pad, 16), jnp.float32),
        ),
        mesh=mesh,
        scratch_types=(
            [pltpu.VMEM_SHARED((n_pad, 64), jnp.float32),
             pltpu.VMEM_SHARED((n_pad, 32), jnp.float32),
             pltpu.VMEM_SHARED((n_pad, 16), jnp.float32),
             pltpu.VMEM((chunk, 16), jnp.float32)]
            + [pltpu.VMEM((chunk,), jnp.int32)] * _NBUF
            + [pltpu.VMEM((chunk,), jnp.int32)] * _NBUF
            + [pltpu.VMEM((chunk, 64), jnp.float32)] * _NBUF
            + [pltpu.VMEM((chunk, 32), jnp.float32)] * _NBUF
            + [pltpu.SemaphoreType.DMA] * (3 * _NBUF)
        ),
    )
    def sc_kernel(token2_h, idx2_h, dstp_h, attr_h, za_h, zb_h, zc_h,
                  acc_h, cnt_h, acc_a, acc_b, acc_c, ones_v, *bufs):
        idx_v = bufs[0:_NBUF]
        dst_v = bufs[_NBUF:2 * _NBUF]
        rows_v = bufs[2 * _NBUF:3 * _NBUF]
        attr_v = bufs[3 * _NBUF:4 * _NBUF]
        sem_i = bufs[4 * _NBUF:5 * _NBUF]
        sem_g = bufs[5 * _NBUF:6 * _NBUF]
        sem_s = bufs[6 * _NBUF:7 * _NBUF]
        c = lax.axis_index("c")
        s = lax.axis_index("s")
        r0 = s * rpt
        rows = pl.ds(r0, rpt)
        pltpu.sync_copy(za_h.at[rows], acc_a.at[rows])
        pltpu.sync_copy(zb_h.at[rows], acc_b.at[rows])
        @pl.when(c == 0)
        def _():
            pltpu.sync_copy(zc_h.at[rows], acc_c.at[rows])

        for j in range(chunk):
            ones_v[j] = jnp.full((16,), 1.0, jnp.float32)
        plsc.subcore_barrier()

        ebase = s * ept
        idx_base = c * ep

        def issue_idx(i, b):
            e0 = ebase + i * chunk
            pltpu.async_copy(idx2_h.at[pl.ds(idx_base + e0, chunk)],
                             idx_v[b], sem_i[b])
            pltpu.async_copy(dstp_h.at[pl.ds(e0, chunk)], dst_v[b], sem_i[b])

        def wait_idx(b):
            pltpu.make_async_copy(idx2_h.at[pl.ds(0, chunk)], idx_v[b],
                                  sem_i[b]).wait()
            pltpu.make_async_copy(dstp_h.at[pl.ds(0, chunk)], dst_v[b],
                                  sem_i[b]).wait()

        def issue_ga(i, b):
            pltpu.async_copy(token2_h.at[idx_v[b]], rows_v[b], sem_g[b])
            a0 = jnp.minimum(ebase + i * chunk, e_real - chunk)
            pltpu.async_copy(attr_h.at[pl.ds(a0, chunk), pl.ds(c * 32, 32)],
                             attr_v[b], sem_g[b])

        def wait_ga(b):
            pltpu.make_async_copy(token2_h.at[idx_v[b]], rows_v[b],
                                  sem_g[b]).wait()
            pltpu.make_async_copy(attr_h.at[pl.ds(0, chunk), pl.ds(0, 32)],
                                  attr_v[b], sem_g[b]).wait()

        def issue_scatter(b):
            pltpu.async_copy(rows_v[b], acc_a.at[dst_v[b]], sem_s[b], add=True)
            pltpu.async_copy(attr_v[b], acc_b.at[dst_v[b]], sem_s[b], add=True)

            @pl.when(c == 0)
            def _():
                pltpu.async_copy(ones_v, acc_c.at[dst_v[b]], sem_s[b],
                                 add=True)

        def wait_scatter(b):
            pltpu.make_async_copy(rows_v[b], acc_a.at[dst_v[b]],
                                  sem_s[b]).wait()
            pltpu.make_async_copy(attr_v[b], acc_b.at[dst_v[b]],
                                  sem_s[b]).wait()

            @pl.when(c == 0)
            def _():
                pltpu.make_async_copy(ones_v, acc_c.at[dst_v[b]],
                                      sem_s[b]).wait()

        issue_idx(0, 0)
        issue_idx(1, 1)
        issue_idx(2, 2)
        wait_idx(0)
        issue_ga(0, 0)

        def body(k, carry):
            for b in range(_NBUF):
                i = 4 * k + b
                b_prev = (b - 1) % _NBUF
                b_next = (b + 1) % _NBUF

                @pl.when(i >= 1)
                def _():
                    wait_scatter(b_prev)

                @pl.when(i + 3 < n_chunks)
                def _():
                    issue_idx(i + 3, b_prev)

                @pl.when(i + 1 < n_chunks)
                def _():
                    wait_idx(b_next)
                    issue_ga(i + 1, b_next)

                wait_ga(b)
                issue_scatter(b)

            return carry

        lax.fori_loop(0, n_chunks // _NBUF, body, 0)
        wait_scatter((n_chunks - 1) % _NBUF)
        plsc.subcore_barrier()

        pltpu.sync_copy(acc_a.at[rows], acc_h.at[rows, pl.ds(c * 64, 64)])
        pltpu.sync_copy(acc_b.at[rows], acc_h.at[rows, pl.ds(128 + c * 32, 32)])

        @pl.when(c == 0)
        def _():
            pltpu.sync_copy(acc_c.at[rows], cnt_h.at[rows])

    return sc_kernel(token2, idx2, dstp, attr, z_a, z_b, z_c)


def _tc_body(acc_ref, cnt_ref, row_ref, w_ref, b_ref, g_ref, be_ref, out_ref):
    s = jnp.dot(acc_ref[...], w_ref[...], preferred_element_type=jnp.float32)
    cnt = cnt_ref[:, 0:1]
    msg = (s + cnt * b_ref[...]) / jnp.maximum(cnt, 1.0)
    x = row_ref[...] + msg
    mu = jnp.mean(x, axis=-1, keepdims=True)
    var = jnp.mean((x - mu) ** 2, axis=-1, keepdims=True)
    out_ref[...] = (x - mu) * lax.rsqrt(var + 1e-5) * g_ref[...] + be_ref[...]


def kernel(row_x, token_x, t2r_edge_index, edge_attr_t2r, r2t_edge_index,
           edge_attr_r2t, W, b, gamma, beta):
    n_rows, d = row_x.shape
    de = edge_attr_t2r.shape[1]
    e = t2r_edge_index.shape[1]
    assert e % _CHUNK == 0
    n_pad = 10240

    shard = _NSUB * _CHUNK * _NBUF
    ep = -(-e // shard) * shard
    pad = ep - e
    src = jnp.concatenate([t2r_edge_index[0], jnp.zeros((pad,), jnp.int32)])
    idx2 = jnp.concatenate([src * 2, src * 2 + 1])
    dstp = jnp.concatenate([t2r_edge_index[1],
                            jnp.full((pad,), n_rows, jnp.int32)])

    token2 = token_x.reshape(-1, d // 2)
    z_a = jnp.zeros((n_pad, 64), jnp.float32)
    z_b = jnp.zeros((n_pad, 32), jnp.float32)
    z_c = jnp.zeros((n_pad, 16), jnp.float32)
    acc, cnt = _sc_accumulate(token2, idx2, dstp, edge_attr_t2r,
                              z_a, z_b, z_c)

    blk = 1024
    grid = -(-n_rows // blk)
    row_new = pl.pallas_call(
        _tc_body,
        grid=(grid,),
        in_specs=[
            pl.BlockSpec((blk, d + de), lambda i: (i, 0)),
            pl.BlockSpec((blk, 16), lambda i: (i, 0)),
            pl.BlockSpec((blk, d), lambda i: (i, 0)),
            pl.BlockSpec((d + de, d), lambda i: (0, 0)),
            pl.BlockSpec((1, d), lambda i: (0, 0)),
            pl.BlockSpec((1, d), lambda i: (0, 0)),
            pl.BlockSpec((1, d), lambda i: (0, 0)),
        ],
        out_specs=pl.BlockSpec((blk, d), lambda i: (i, 0)),
        out_shape=jax.ShapeDtypeStruct((n_rows, d), jnp.float32),
    )(acc, cnt, row_x, W, b.reshape(1, d), gamma.reshape(1, d),
      beta.reshape(1, d))
    return (row_new, token_x)


# speedup vs baseline: 1.0022x; 1.0022x over previous
import functools

import jax
import jax.numpy as jnp
from jax import lax
from jax.experimental import pallas as pl
from jax.experimental.pallas import tpu as pltpu
from jax.experimental.pallas import tpu_sc as plsc

_CHUNK = 128
_NSUB = 16
_NBUF = 4


def _sc_accumulate(token2, idx2, dstp, attr, z_a, z_b, z_c):
    n_pad = z_a.shape[0]
    ep = dstp.shape[0]
    e_real = attr.shape[0]
    chunk = _CHUNK
    ept = ep // _NSUB
    n_chunks = ept // chunk
    assert ept % chunk == 0 and n_chunks % _NBUF == 0
    assert n_pad % (8 * _NSUB) == 0
    rpt = n_pad // _NSUB

    mesh = plsc.VectorSubcoreMesh(core_axis_name="c", subcore_axis_name="s")

    @functools.partial(
        pl.kernel,
        compiler_params=pltpu.CompilerParams(use_tc_tiling_on_sc=False),
        out_type=(
            jax.ShapeDtypeStruct((n_pad, 192), jnp.float32),
            jax.ShapeDtypeStruct((n_pad, 16), jnp.float32),
        ),
        mesh=mesh,
        scratch_types=(
            [pltpu.VMEM_SHARED((n_pad, 64), jnp.float32),
             pltpu.VMEM_SHARED((n_pad, 32), jnp.float32),
             pltpu.VMEM_SHARED((n_pad, 16), jnp.float32),
             pltpu.VMEM((chunk, 16), jnp.float32)]
            + [pltpu.VMEM((chunk,), jnp.int32)] * _NBUF
            + [pltpu.VMEM((chunk,), jnp.int32)] * _NBUF
            + [pltpu.VMEM((chunk, 64), jnp.float32)] * _NBUF
            + [pltpu.VMEM((chunk, 32), jnp.float32)] * _NBUF
            + [pltpu.SemaphoreType.DMA] * (3 * _NBUF)
        ),
    )
    def sc_kernel(token2_h, idx2_h, dstp_h, attr_h, za_h, zb_h, zc_h,
                  acc_h, cnt_h, acc_a, acc_b, acc_c, ones_v, *bufs):
        idx_v = bufs[0:_NBUF]
        dst_v = bufs[_NBUF:2 * _NBUF]
        rows_v = bufs[2 * _NBUF:3 * _NBUF]
        attr_v = bufs[3 * _NBUF:4 * _NBUF]
        sem_i = bufs[4 * _NBUF:5 * _NBUF]
        sem_g = bufs[5 * _NBUF:6 * _NBUF]
        sem_s = bufs[6 * _NBUF:7 * _NBUF]
        c = lax.axis_index("c")
        s = lax.axis_index("s")
        r0 = s * rpt
        rows = pl.ds(r0, rpt)
        pltpu.sync_copy(za_h.at[rows], acc_a.at[rows])
        pltpu.sync_copy(zb_h.at[rows], acc_b.at[rows])
        @pl.when(c == 0)
        def _():
            pltpu.sync_copy(zc_h.at[rows], acc_c.at[rows])

        for j in range(chunk):
            ones_v[j] = jnp.full((16,), 1.0, jnp.float32)
        plsc.subcore_barrier()

        ebase = s * ept
        idx_base = c * ep

        def issue_idx(i, b):
            e0 = ebase + i * chunk
            pltpu.async_copy(idx2_h.at[pl.ds(idx_base + e0, chunk)],
                             idx_v[b], sem_i[b])
            pltpu.async_copy(dstp_h.at[pl.ds(e0, chunk)], dst_v[b], sem_i[b])

        def wait_idx(b):
            pltpu.make_async_copy(idx2_h.at[pl.ds(0, chunk)], idx_v[b],
                                  sem_i[b]).wait()
            pltpu.make_async_copy(dstp_h.at[pl.ds(0, chunk)], dst_v[b],
                                  sem_i[b]).wait()

        def issue_ga(i, b):
            pltpu.async_copy(token2_h.at[idx_v[b]], rows_v[b], sem_g[b])
            a0 = jnp.minimum(ebase + i * chunk, e_real - chunk)
            pltpu.async_copy(attr_h.at[pl.ds(a0, chunk), pl.ds(c * 32, 32)],
                             attr_v[b], sem_g[b])

        def wait_ga(b):
            pltpu.make_async_copy(token2_h.at[idx_v[b]], rows_v[b],
                                  sem_g[b]).wait()
            pltpu.make_async_copy(attr_h.at[pl.ds(0, chunk), pl.ds(0, 32)],
                                  attr_v[b], sem_g[b]).wait()

        def issue_scatter(b):
            pltpu.async_copy(rows_v[b], acc_a.at[dst_v[b]], sem_s[b], add=True)
            pltpu.async_copy(attr_v[b], acc_b.at[dst_v[b]], sem_s[b], add=True)

            @pl.when(c == 0)
            def _():
                pltpu.async_copy(ones_v, acc_c.at[dst_v[b]], sem_s[b],
                                 add=True)

        def wait_scatter(b):
            pltpu.make_async_copy(rows_v[b], acc_a.at[dst_v[b]],
                                  sem_s[b]).wait()
            pltpu.make_async_copy(attr_v[b], acc_b.at[dst_v[b]],
                                  sem_s[b]).wait()

            @pl.when(c == 0)
            def _():
                pltpu.make_async_copy(ones_v, acc_c.at[dst_v[b]],
                                      sem_s[b]).wait()

        issue_idx(0, 0)
        issue_idx(1, 1)
        issue_idx(2, 2)
        wait_idx(0)
        issue_ga(0, 0)

        def body(k, carry):
            for b in range(_NBUF):
                i = 4 * k + b
                b_prev = (b - 1) % _NBUF
                b_next = (b + 1) % _NBUF

                @pl.when(i >= 1)
                def _():
                    wait_scatter(b_prev)

                @pl.when(i + 3 < n_chunks)
                def _():
                    issue_idx(i + 3, b_prev)

                @pl.when(i + 1 < n_chunks)
                def _():
                    wait_idx(b_next)
                    issue_ga(i + 1, b_next)

                wait_ga(b)
                issue_scatter(b)

            return carry

        lax.fori_loop(0, n_chunks // _NBUF, body, 0)
        wait_scatter((n_chunks - 1) % _NBUF)
        plsc.subcore_barrier()

        pltpu.sync_copy(acc_a.at[rows], acc_h.at[rows, pl.ds(c * 64, 64)])
        pltpu.sync_copy(acc_b.at[rows], acc_h.at[rows, pl.ds(128 + c * 32, 32)])

        @pl.when(c == 0)
        def _():
            pltpu.sync_copy(acc_c.at[rows], cnt_h.at[rows])

    return sc_kernel(token2, idx2, dstp, attr, z_a, z_b, z_c)


def _tc_body(acc_ref, cnt_ref, row_ref, w_ref, b_ref, g_ref, be_ref, out_ref):
    s = jnp.dot(acc_ref[...], w_ref[...], preferred_element_type=jnp.float32)
    cnt = cnt_ref[:, 0:1]
    msg = (s + cnt * b_ref[...]) / jnp.maximum(cnt, 1.0)
    x = row_ref[...] + msg
    mu = jnp.mean(x, axis=-1, keepdims=True)
    var = jnp.mean((x - mu) ** 2, axis=-1, keepdims=True)
    out_ref[...] = (x - mu) * lax.rsqrt(var + 1e-5) * g_ref[...] + be_ref[...]


def kernel(row_x, token_x, t2r_edge_index, edge_attr_t2r, r2t_edge_index,
           edge_attr_r2t, W, b, gamma, beta):
    n_rows, d = row_x.shape
    de = edge_attr_t2r.shape[1]
    e = t2r_edge_index.shape[1]
    assert e % _CHUNK == 0
    n_pad = 10240

    shard = _NSUB * _CHUNK * _NBUF
    ep = -(-e // shard) * shard
    pad = ep - e
    src = jnp.concatenate([t2r_edge_index[0], jnp.zeros((pad,), jnp.int32)])
    idx2 = jnp.concatenate([src * 2, src * 2 + 1])
    dstp = jnp.concatenate([t2r_edge_index[1],
                            jnp.full((pad,), n_rows, jnp.int32)])

    token2 = token_x.reshape(-1, d // 2)
    z_a = jnp.zeros((n_pad, 64), jnp.float32)
    z_b = jnp.zeros((n_pad, 32), jnp.float32)
    z_c = jnp.zeros((n_pad, 16), jnp.float32)
    acc, cnt = _sc_accumulate(token2, idx2, dstp, edge_attr_t2r,
                              z_a, z_b, z_c)

    blk = 1024
    grid = -(-n_rows // blk)
    row_new = pl.pallas_call(
        _tc_body,
        grid=(grid,),
        in_specs=[
            pl.BlockSpec((blk, d + de), lambda i: (i, 0)),
            pl.BlockSpec((blk, 16), lambda i: (i, 0)),
            pl.BlockSpec((blk, d), lambda i: (i, 0)),
            pl.BlockSpec((d + de, d), lambda i: (0, 0)),
            pl.BlockSpec((1, d), lambda i: (0, 0)),
            pl.BlockSpec((1, d), lambda i: (0, 0)),
            pl.BlockSpec((1, d), lambda i: (0, 0)),
        ],
        out_specs=pl.BlockSpec((blk, d), lambda i: (i, 0)),
        out_shape=jax.ShapeDtypeStruct((n_rows, d), jnp.float32),
    )(acc, cnt, row_x, W, b.reshape(1, d), gamma.reshape(1, d),
      beta.reshape(1, d))
    return (row_new, token_x)


# speedup vs baseline: 1.2177x; 1.2151x over previous
import functools

import jax
import jax.numpy as jnp
from jax import lax
from jax.experimental import pallas as pl
from jax.experimental.pallas import tpu as pltpu
from jax.experimental.pallas import tpu_sc as plsc

_CHUNK = 128
_NSUB = 16


def _sc_accumulate(token2, idx2, dstp, attr, z_a, z_b, z_c):
    n_pad = z_a.shape[0]
    ep = dstp.shape[0]
    e_real = attr.shape[0]
    chunk = _CHUNK
    ept = ep // _NSUB
    n_chunks = ept // chunk
    assert ept % chunk == 0 and n_chunks % 2 == 0 and n_pad % (8 * _NSUB) == 0
    rpt = n_pad // _NSUB

    mesh = plsc.VectorSubcoreMesh(core_axis_name="c", subcore_axis_name="s")

    @functools.partial(
        pl.kernel,
        compiler_params=pltpu.CompilerParams(use_tc_tiling_on_sc=False),
        out_type=(
            jax.ShapeDtypeStruct((n_pad, 192), jnp.float32),
            jax.ShapeDtypeStruct((n_pad, 16), jnp.float32),
        ),
        mesh=mesh,
        scratch_types=[
            pltpu.VMEM_SHARED((n_pad, 64), jnp.float32),
            pltpu.VMEM_SHARED((n_pad, 32), jnp.float32),
            pltpu.VMEM_SHARED((n_pad, 16), jnp.float32),
            pltpu.VMEM((chunk,), jnp.int32),
            pltpu.VMEM((chunk,), jnp.int32),
            pltpu.VMEM((chunk,), jnp.int32),
            pltpu.VMEM((chunk,), jnp.int32),
            pltpu.VMEM((chunk, 64), jnp.float32),
            pltpu.VMEM((chunk, 64), jnp.float32),
            pltpu.VMEM((chunk, 32), jnp.float32),
            pltpu.VMEM((chunk, 32), jnp.float32),
            pltpu.VMEM((chunk, 16), jnp.float32),
            pltpu.SemaphoreType.DMA,
            pltpu.SemaphoreType.DMA,
            pltpu.SemaphoreType.DMA,
            pltpu.SemaphoreType.DMA,
            pltpu.SemaphoreType.DMA,
            pltpu.SemaphoreType.DMA,
            pltpu.SemaphoreType.DMA,
            pltpu.SemaphoreType.DMA,
        ],
    )
    def sc_kernel(token2_h, idx2_h, dstp_h, attr_h, za_h, zb_h, zc_h,
                  acc_h, cnt_h, acc_a, acc_b, acc_c,
                  idx_v0, idx_v1, dst_v0, dst_v1, rows_v0, rows_v1,
                  attr_v0, attr_v1, ones_v,
                  sem_i0, sem_i1, sem_g0, sem_g1, sem_a0, sem_a1,
                  sem_s0, sem_s1):
        c = lax.axis_index("c")
        s = lax.axis_index("s")
        idx_v = (idx_v0, idx_v1)
        dst_v = (dst_v0, dst_v1)
        rows_v = (rows_v0, rows_v1)
        attr_v = (attr_v0, attr_v1)
        sem_i = (sem_i0, sem_i1)
        sem_g = (sem_g0, sem_g1)
        sem_a = (sem_a0, sem_a1)
        sem_s = (sem_s0, sem_s1)

        r0 = s * rpt
        rows = pl.ds(r0, rpt)
        pltpu.sync_copy(za_h.at[rows], acc_a.at[rows])
        pltpu.sync_copy(zb_h.at[rows], acc_b.at[rows])

        @pl.when(c == 0)
        def _():
            pltpu.sync_copy(zc_h.at[rows], acc_c.at[rows])

        for j in range(chunk):
            ones_v[j] = jnp.full((16,), 1.0, jnp.float32)
        plsc.subcore_barrier()

        ebase = s * ept
        idx_base = c * ep

        def issue_idx(i, b):
            e0 = ebase + i * chunk
            pltpu.async_copy(idx2_h.at[pl.ds(idx_base + e0, chunk)],
                             idx_v[b], sem_i[b])
            pltpu.async_copy(dstp_h.at[pl.ds(e0, chunk)], dst_v[b], sem_i[b])

        def wait_idx(b):
            pltpu.make_async_copy(idx2_h.at[pl.ds(0, chunk)], idx_v[b],
                                  sem_i[b]).wait()
            pltpu.make_async_copy(dstp_h.at[pl.ds(0, chunk)], dst_v[b],
                                  sem_i[b]).wait()

        def issue_ga(i, b):
            pltpu.async_copy(token2_h.at[idx_v[b]], rows_v[b], sem_g[b])
            a0 = jnp.minimum(ebase + i * chunk, e_real - chunk)
            pltpu.async_copy(attr_h.at[pl.ds(a0, chunk), pl.ds(c * 32, 32)],
                             attr_v[b], sem_a[b])

        def wait_ga(b):
            pltpu.make_async_copy(token2_h.at[idx_v[b]], rows_v[b],
                                  sem_g[b]).wait()
            pltpu.make_async_copy(attr_h.at[pl.ds(0, chunk), pl.ds(0, 32)],
                                  attr_v[b], sem_a[b]).wait()

        def issue_scatter(b):
            pltpu.async_copy(rows_v[b], acc_a.at[dst_v[b]], sem_s[b], add=True)
            pltpu.async_copy(attr_v[b], acc_b.at[dst_v[b]], sem_s[b], add=True)

            @pl.when(c == 0)
            def _():
                pltpu.async_copy(ones_v, acc_c.at[dst_v[b]], sem_s[b],
                                 add=True)

        def wait_scatter(b):
            pltpu.make_async_copy(rows_v[b], acc_a.at[dst_v[b]],
                                  sem_s[b]).wait()
            pltpu.make_async_copy(attr_v[b], acc_b.at[dst_v[b]],
                                  sem_s[b]).wait()

            @pl.when(c == 0)
            def _():
                pltpu.make_async_copy(ones_v, acc_c.at[dst_v[b]],
                                      sem_s[b]).wait()

        issue_idx(0, 0)
        wait_idx(0)
        issue_ga(0, 0)
        issue_idx(1, 1)

        def body(k, carry):
            for b in (0, 1):
                i = 2 * k + b
                b1 = 1 - b

                @pl.when(i + 1 < n_chunks)
                def _():
                    wait_idx(b1)
                    issue_ga(i + 1, b1)

                wait_ga(b)
                issue_scatter(b)
                wait_scatter(b)

                @pl.when(i + 2 < n_chunks)
                def _():
                    issue_idx(i + 2, b)

            return carry

        lax.fori_loop(0, n_chunks // 2, body, 0)
        plsc.subcore_barrier()

        pltpu.sync_copy(acc_a.at[rows], acc_h.at[rows, pl.ds(c * 64, 64)])
        pltpu.sync_copy(acc_b.at[rows], acc_h.at[rows, pl.ds(128 + c * 32, 32)])

        @pl.when(c == 0)
        def _():
            pltpu.sync_copy(acc_c.at[rows], cnt_h.at[rows])

    return sc_kernel(token2, idx2, dstp, attr, z_a, z_b, z_c)


def _tc_body(acc_ref, cnt_ref, row_ref, w_ref, b_ref, g_ref, be_ref, out_ref):
    s = jnp.dot(acc_ref[...], w_ref[...], preferred_element_type=jnp.float32)
    cnt = cnt_ref[:, 0:1]
    msg = (s + cnt * b_ref[...]) / jnp.maximum(cnt, 1.0)
    x = row_ref[...] + msg
    mu = jnp.mean(x, axis=-1, keepdims=True)
    var = jnp.mean((x - mu) ** 2, axis=-1, keepdims=True)
    out_ref[...] = (x - mu) * lax.rsqrt(var + 1e-5) * g_ref[...] + be_ref[...]


def kernel(row_x, token_x, t2r_edge_index, edge_attr_t2r, r2t_edge_index,
           edge_attr_r2t, W, b, gamma, beta):
    n_rows, d = row_x.shape
    de = edge_attr_t2r.shape[1]
    e = t2r_edge_index.shape[1]
    assert e % _CHUNK == 0
    n_pad = 10240

    shard = 2 * _NSUB * _CHUNK
    ep = -(-e // shard) * shard
    pad = ep - e
    src = jnp.concatenate([t2r_edge_index[0], jnp.zeros((pad,), jnp.int32)])
    idx2 = jnp.concatenate([src * 2, src * 2 + 1])
    dstp = jnp.concatenate([t2r_edge_index[1],
                            jnp.full((pad,), n_rows, jnp.int32)])

    token2 = token_x.reshape(-1, d // 2)
    z_a = jnp.zeros((n_pad, 64), jnp.float32)
    z_b = jnp.zeros((n_pad, 32), jnp.float32)
    z_c = jnp.zeros((n_pad, 16), jnp.float32)
    acc, cnt = _sc_accumulate(token2, idx2, dstp, edge_attr_t2r,
                              z_a, z_b, z_c)

    blk = 1024
    grid = -(-n_rows // blk)
    row_new = pl.pallas_call(
        _tc_body,
        grid=(grid,),
        in_specs=[
            pl.BlockSpec((blk, d + de), lambda i: (i, 0)),
            pl.BlockSpec((blk, 16), lambda i: (i, 0)),
            pl.BlockSpec((blk, d), lambda i: (i, 0)),
            pl.BlockSpec((d + de, d), lambda i: (0, 0)),
            pl.BlockSpec((1, d), lambda i: (0, 0)),
            pl.BlockSpec((1, d), lambda i: (0, 0)),
            pl.BlockSpec((1, d), lambda i: (0, 0)),
        ],
        out_specs=pl.BlockSpec((blk, d), lambda i: (i, 0)),
        out_shape=jax.ShapeDtypeStruct((n_rows, d), jnp.float32),
    )(acc, cnt, row_x, W, b.reshape(1, d), gamma.reshape(1, d),
      beta.reshape(1, d))
    return (row_new, token_x)


# speedup vs baseline: 1.5452x; 1.2689x over previous
import functools

import jax
import jax.numpy as jnp
from jax import lax
from jax.experimental import pallas as pl
from jax.experimental.pallas import tpu as pltpu
from jax.experimental.pallas import tpu_sc as plsc

_CHUNK = 128
_NSUB = 16


def _sc_accumulate(token2, edges, attr, z_a, z_b, z_c):
    n_pad = z_a.shape[0]
    e_real = attr.shape[0]
    chunk = _CHUNK
    total_chunks = e_real // chunk
    nc0 = (total_chunks // _NSUB) & ~1
    rem = total_chunks - (_NSUB - 1) * nc0
    assert e_real % chunk == 0 and e_real % 8 == 0 and rem % 2 == 0 and rem > 0
    assert n_pad % (8 * _NSUB) == 0
    rpt = n_pad // _NSUB

    mesh = plsc.VectorSubcoreMesh(core_axis_name="c", subcore_axis_name="s")

    @functools.partial(
        pl.kernel,
        compiler_params=pltpu.CompilerParams(use_tc_tiling_on_sc=False),
        out_type=(
            jax.ShapeDtypeStruct((n_pad, 192), jnp.float32),
            jax.ShapeDtypeStruct((n_pad, 16), jnp.float32),
        ),
        mesh=mesh,
        scratch_types=[
            pltpu.VMEM_SHARED((n_pad, 64), jnp.float32),
            pltpu.VMEM_SHARED((n_pad, 32), jnp.float32),
            pltpu.VMEM_SHARED((n_pad, 16), jnp.float32),
            pltpu.VMEM((chunk,), jnp.int32),
            pltpu.VMEM((chunk,), jnp.int32),
            pltpu.VMEM((chunk,), jnp.int32),
            pltpu.VMEM((chunk,), jnp.int32),
            pltpu.VMEM((chunk,), jnp.int32),
            pltpu.VMEM((chunk,), jnp.int32),
            pltpu.VMEM((chunk, 64), jnp.float32),
            pltpu.VMEM((chunk, 64), jnp.float32),
            pltpu.VMEM((chunk, 32), jnp.float32),
            pltpu.VMEM((chunk, 32), jnp.float32),
            pltpu.VMEM((chunk, 16), jnp.float32),
            pltpu.SemaphoreType.DMA,
            pltpu.SemaphoreType.DMA,
            pltpu.SemaphoreType.DMA,
            pltpu.SemaphoreType.DMA,
            pltpu.SemaphoreType.DMA,
            pltpu.SemaphoreType.DMA,
            pltpu.SemaphoreType.DMA,
            pltpu.SemaphoreType.DMA,
        ],
    )
    def sc_kernel(token2_h, edges_h, attr_h, za_h, zb_h, zc_h,
                  acc_h, cnt_h, acc_a, acc_b, acc_c,
                  idx_v0, idx_v1, src_v0, src_v1, dst_v0, dst_v1,
                  rows_v0, rows_v1, attr_v0, attr_v1, ones_v,
                  sem_i0, sem_i1, sem_g0, sem_g1, sem_a0, sem_a1,
                  sem_s0, sem_s1):
        c = lax.axis_index("c")
        s = lax.axis_index("s")
        idx_v = (idx_v0, idx_v1)
        src_v = (src_v0, src_v1)
        dst_v = (dst_v0, dst_v1)
        rows_v = (rows_v0, rows_v1)
        attr_v = (attr_v0, attr_v1)
        sem_i = (sem_i0, sem_i1)
        sem_g = (sem_g0, sem_g1)
        sem_a = (sem_a0, sem_a1)
        sem_s = (sem_s0, sem_s1)

        r0 = s * rpt
        rows = pl.ds(r0, rpt)
        pltpu.sync_copy(za_h.at[rows], acc_a.at[rows])
        pltpu.sync_copy(zb_h.at[rows], acc_b.at[rows])

        @pl.when(c == 0)
        def _():
            pltpu.sync_copy(zc_h.at[rows], acc_c.at[rows])

        for j in range(chunk):
            ones_v[j] = jnp.full((16,), 1.0, jnp.float32)
        plsc.subcore_barrier()

        ebase = s * nc0 * chunk
        nc = jnp.where(s == _NSUB - 1, rem, nc0)

        def issue_idx(i, b):
            e0 = ebase + i * chunk
            pltpu.async_copy(edges_h.at[0, pl.ds(e0, chunk)], src_v[b],
                             sem_i[b])
            pltpu.async_copy(edges_h.at[1, pl.ds(e0, chunk)], dst_v[b],
                             sem_i[b])

        def wait_idx(b):
            pltpu.make_async_copy(edges_h.at[0, pl.ds(0, chunk)], src_v[b],
                                  sem_i[b]).wait()
            pltpu.make_async_copy(edges_h.at[0, pl.ds(0, chunk)], dst_v[b],
                                  sem_i[b]).wait()
            for j in range(chunk // 16):
                sl = pl.ds(j * 16, 16)
                idx_v[b][sl] = src_v[b][sl] * 2 + c

        def issue_ga(i, b):
            pltpu.async_copy(token2_h.at[idx_v[b]], rows_v[b], sem_g[b])
            a0 = ebase + i * chunk
            pltpu.async_copy(attr_h.at[pl.ds(a0, chunk), pl.ds(c * 32, 32)],
                             attr_v[b], sem_a[b])

        def wait_ga(b):
            pltpu.make_async_copy(token2_h.at[idx_v[b]], rows_v[b],
                                  sem_g[b]).wait()
            pltpu.make_async_copy(attr_h.at[pl.ds(0, chunk), pl.ds(0, 32)],
                                  attr_v[b], sem_a[b]).wait()

        def issue_scatter(b):
            pltpu.async_copy(rows_v[b], acc_a.at[dst_v[b]], sem_s[b], add=True)
            pltpu.async_copy(attr_v[b], acc_b.at[dst_v[b]], sem_s[b], add=True)

            @pl.when(c == 0)
            def _():
                pltpu.async_copy(ones_v, acc_c.at[dst_v[b]], sem_s[b],
                                 add=True)

        def wait_scatter(b):
            pltpu.make_async_copy(rows_v[b], acc_a.at[dst_v[b]],
                                  sem_s[b]).wait()
            pltpu.make_async_copy(attr_v[b], acc_b.at[dst_v[b]],
                                  sem_s[b]).wait()

            @pl.when(c == 0)
            def _():
                pltpu.make_async_copy(ones_v, acc_c.at[dst_v[b]],
                                      sem_s[b]).wait()

        issue_idx(0, 0)
        wait_idx(0)
        issue_ga(0, 0)
        issue_idx(1, 1)

        def body(k, carry):
            for b in (0, 1):
                i = 2 * k + b
                b1 = 1 - b

                @pl.when(i + 1 < nc)
                def _():
                    wait_idx(b1)
                    issue_ga(i + 1, b1)

                wait_ga(b)
                issue_scatter(b)
                wait_scatter(b)

                @pl.when(i + 2 < nc)
                def _():
                    issue_idx(i + 2, b)

            return carry

        lax.fori_loop(0, nc // 2, body, 0)
        plsc.subcore_barrier()

        pltpu.sync_copy(acc_a.at[rows], acc_h.at[rows, pl.ds(c * 64, 64)])
        pltpu.sync_copy(acc_b.at[rows], acc_h.at[rows, pl.ds(128 + c * 32, 32)])

        @pl.when(c == 0)
        def _():
            pltpu.sync_copy(acc_c.at[rows], cnt_h.at[rows])

    return sc_kernel(token2, edges, attr, z_a, z_b, z_c)


def _tc_body(acc_ref, cnt_ref, row_ref, w_ref, b_ref, g_ref, be_ref, out_ref):
    s = jnp.dot(acc_ref[...], w_ref[...], preferred_element_type=jnp.float32)
    cnt = cnt_ref[:, 0:1]
    msg = (s + cnt * b_ref[...]) / jnp.maximum(cnt, 1.0)
    x = row_ref[...] + msg
    mu = jnp.mean(x, axis=-1, keepdims=True)
    var = jnp.mean((x - mu) ** 2, axis=-1, keepdims=True)
    out_ref[...] = (x - mu) * lax.rsqrt(var + 1e-5) * g_ref[...] + be_ref[...]


def kernel(row_x, token_x, t2r_edge_index, edge_attr_t2r, r2t_edge_index,
           edge_attr_r2t, W, b, gamma, beta):
    n_rows, d = row_x.shape
    de = edge_attr_t2r.shape[1]
    e = t2r_edge_index.shape[1]
    assert e % _CHUNK == 0
    n_pad = 10240

    token2 = token_x.reshape(-1, d // 2)
    z_a = jnp.zeros((n_pad, 64), jnp.float32)
    z_b = jnp.zeros((n_pad, 32), jnp.float32)
    z_c = jnp.zeros((n_pad, 16), jnp.float32)
    acc, cnt = _sc_accumulate(token2, t2r_edge_index, edge_attr_t2r,
                              z_a, z_b, z_c)

    blk = 1024
    grid = -(-n_rows // blk)
    row_new = pl.pallas_call(
        _tc_body,
        grid=(grid,),
        in_specs=[
            pl.BlockSpec((blk, d + de), lambda i: (i, 0)),
            pl.BlockSpec((blk, 16), lambda i: (i, 0)),
            pl.BlockSpec((blk, d), lambda i: (i, 0)),
            pl.BlockSpec((d + de, d), lambda i: (0, 0)),
            pl.BlockSpec((1, d), lambda i: (0, 0)),
            pl.BlockSpec((1, d), lambda i: (0, 0)),
            pl.BlockSpec((1, d), lambda i: (0, 0)),
        ],
        out_specs=pl.BlockSpec((blk, d), lambda i: (i, 0)),
        out_shape=jax.ShapeDtypeStruct((n_rows, d), jnp.float32),
    )(acc, cnt, row_x, W, b.reshape(1, d), gamma.reshape(1, d),
      beta.reshape(1, d))
    return (row_new, token_x)


# speedup vs baseline: 1.5516x; 1.0041x over previous
import functools

import jax
import jax.numpy as jnp
from jax import lax
from jax.experimental import pallas as pl
from jax.experimental.pallas import tpu as pltpu
from jax.experimental.pallas import tpu_sc as plsc

_CHUNK = 128
_NSUB = 16


def _sc_accumulate(token2, edges, attr, n_pad):
    e_real = attr.shape[0]
    chunk = _CHUNK
    total_chunks = e_real // chunk
    nc0 = (total_chunks // _NSUB) & ~1
    rem = total_chunks - (_NSUB - 1) * nc0
    assert e_real % chunk == 0 and e_real % 8 == 0 and rem % 2 == 0 and rem > 0
    assert n_pad % (8 * _NSUB) == 0
    rpt = n_pad // _NSUB

    mesh = plsc.VectorSubcoreMesh(core_axis_name="c", subcore_axis_name="s")

    @functools.partial(
        pl.kernel,
        compiler_params=pltpu.CompilerParams(use_tc_tiling_on_sc=False),
        out_type=(
            jax.ShapeDtypeStruct((n_pad, 192), jnp.float32),
            jax.ShapeDtypeStruct((n_pad, 16), jnp.float32),
        ),
        mesh=mesh,
        scratch_types=[
            pltpu.VMEM_SHARED((n_pad, 64), jnp.float32),
            pltpu.VMEM_SHARED((n_pad, 32), jnp.float32),
            pltpu.VMEM_SHARED((n_pad, 16), jnp.float32),
            pltpu.VMEM((chunk,), jnp.int32),
            pltpu.VMEM((chunk,), jnp.int32),
            pltpu.VMEM((chunk,), jnp.int32),
            pltpu.VMEM((chunk,), jnp.int32),
            pltpu.VMEM((chunk,), jnp.int32),
            pltpu.VMEM((chunk,), jnp.int32),
            pltpu.VMEM((chunk, 64), jnp.float32),
            pltpu.VMEM((chunk, 64), jnp.float32),
            pltpu.VMEM((chunk, 32), jnp.float32),
            pltpu.VMEM((chunk, 32), jnp.float32),
            pltpu.VMEM((chunk, 16), jnp.float32),
            pltpu.SemaphoreType.DMA,
            pltpu.SemaphoreType.DMA,
            pltpu.SemaphoreType.DMA,
            pltpu.SemaphoreType.DMA,
            pltpu.SemaphoreType.DMA,
            pltpu.SemaphoreType.DMA,
            pltpu.SemaphoreType.DMA,
            pltpu.SemaphoreType.DMA,
        ],
    )
    def sc_kernel(token2_h, edges_h, attr_h, acc_h, cnt_h, acc_a, acc_b, acc_c,
                  idx_v0, idx_v1, src_v0, src_v1, dst_v0, dst_v1,
                  rows_v0, rows_v1, attr_v0, attr_v1, ones_v,
                  sem_i0, sem_i1, sem_g0, sem_g1, sem_a0, sem_a1,
                  sem_s0, sem_s1):
        c = lax.axis_index("c")
        s = lax.axis_index("s")
        idx_v = (idx_v0, idx_v1)
        src_v = (src_v0, src_v1)
        dst_v = (dst_v0, dst_v1)
        rows_v = (rows_v0, rows_v1)
        attr_v = (attr_v0, attr_v1)
        sem_i = (sem_i0, sem_i1)
        sem_g = (sem_g0, sem_g1)
        sem_a = (sem_a0, sem_a1)
        sem_s = (sem_s0, sem_s1)

        r0 = s * rpt
        rows = pl.ds(r0, rpt)

        def zbody(i, carry):
            for jj in range(4):
                rows_v0[i, pl.ds(jj * 16, 16)] = jnp.zeros((16,), jnp.float32)
            return carry

        lax.fori_loop(0, chunk, zbody, 0)
        for r in range(rpt // chunk):
            dst = pl.ds(r0 + r * chunk, chunk)
            pltpu.sync_copy(rows_v0.at[pl.ds(0, chunk)], acc_a.at[dst])
            pltpu.sync_copy(rows_v0.at[pl.ds(0, chunk), pl.ds(0, 32)],
                            acc_b.at[dst])

            @pl.when(c == 0)
            def _():
                pltpu.sync_copy(rows_v0.at[pl.ds(0, chunk), pl.ds(0, 16)],
                                acc_c.at[dst])

        for j in range(chunk):
            ones_v[j] = jnp.full((16,), 1.0, jnp.float32)
        plsc.subcore_barrier()

        ebase = s * nc0 * chunk
        nc = jnp.where(s == _NSUB - 1, rem, nc0)

        def issue_idx(i, b):
            e0 = ebase + i * chunk
            pltpu.async_copy(edges_h.at[0, pl.ds(e0, chunk)], src_v[b],
                             sem_i[b])
            pltpu.async_copy(edges_h.at[1, pl.ds(e0, chunk)], dst_v[b],
                             sem_i[b])

        def wait_idx(b):
            pltpu.make_async_copy(edges_h.at[0, pl.ds(0, chunk)], src_v[b],
                                  sem_i[b]).wait()
            pltpu.make_async_copy(edges_h.at[0, pl.ds(0, chunk)], dst_v[b],
                                  sem_i[b]).wait()
            for j in range(chunk // 16):
                sl = pl.ds(j * 16, 16)
                idx_v[b][sl] = src_v[b][sl] * 2 + c

        def issue_ga(i, b):
            pltpu.async_copy(token2_h.at[idx_v[b]], rows_v[b], sem_g[b])
            a0 = ebase + i * chunk
            pltpu.async_copy(attr_h.at[pl.ds(a0, chunk), pl.ds(c * 32, 32)],
                             attr_v[b], sem_a[b])

        def wait_ga(b):
            pltpu.make_async_copy(token2_h.at[idx_v[b]], rows_v[b],
                                  sem_g[b]).wait()
            pltpu.make_async_copy(attr_h.at[pl.ds(0, chunk), pl.ds(0, 32)],
                                  attr_v[b], sem_a[b]).wait()

        def issue_scatter(b):
            pltpu.async_copy(rows_v[b], acc_a.at[dst_v[b]], sem_s[b], add=True)
            pltpu.async_copy(attr_v[b], acc_b.at[dst_v[b]], sem_s[b], add=True)

            @pl.when(c == 0)
            def _():
                pltpu.async_copy(ones_v, acc_c.at[dst_v[b]], sem_s[b],
                                 add=True)

        def wait_scatter(b):
            pltpu.make_async_copy(rows_v[b], acc_a.at[dst_v[b]],
                                  sem_s[b]).wait()
            pltpu.make_async_copy(attr_v[b], acc_b.at[dst_v[b]],
                                  sem_s[b]).wait()

            @pl.when(c == 0)
            def _():
                pltpu.make_async_copy(ones_v, acc_c.at[dst_v[b]],
                                      sem_s[b]).wait()

        issue_idx(0, 0)
        wait_idx(0)
        issue_ga(0, 0)
        issue_idx(1, 1)

        def body(k, carry):
            for b in (0, 1):
                i = 2 * k + b
                b1 = 1 - b

                @pl.when(i + 1 < nc)
                def _():
                    wait_idx(b1)
                    issue_ga(i + 1, b1)

                wait_ga(b)
                issue_scatter(b)
                wait_scatter(b)

                @pl.when(i + 2 < nc)
                def _():
                    issue_idx(i + 2, b)

            return carry

        lax.fori_loop(0, nc // 2, body, 0)
        plsc.subcore_barrier()

        pltpu.sync_copy(acc_a.at[rows], acc_h.at[rows, pl.ds(c * 64, 64)])
        pltpu.sync_copy(acc_b.at[rows], acc_h.at[rows, pl.ds(128 + c * 32, 32)])

        @pl.when(c == 0)
        def _():
            pltpu.sync_copy(acc_c.at[rows], cnt_h.at[rows])

    return sc_kernel(token2, edges, attr)


def _tc_body(acc_ref, cnt_ref, row_ref, w_ref, b_ref, g_ref, be_ref, out_ref):
    s = jnp.dot(acc_ref[...], w_ref[...], preferred_element_type=jnp.float32)
    cnt = cnt_ref[:, 0:1]
    msg = (s + cnt * b_ref[...]) / jnp.maximum(cnt, 1.0)
    x = row_ref[...] + msg
    mu = jnp.mean(x, axis=-1, keepdims=True)
    var = jnp.mean((x - mu) ** 2, axis=-1, keepdims=True)
    out_ref[...] = (x - mu) * lax.rsqrt(var + 1e-5) * g_ref[...] + be_ref[...]


def kernel(row_x, token_x, t2r_edge_index, edge_attr_t2r, r2t_edge_index,
           edge_attr_r2t, W, b, gamma, beta):
    n_rows, d = row_x.shape
    de = edge_attr_t2r.shape[1]
    e = t2r_edge_index.shape[1]
    assert e % _CHUNK == 0
    n_pad = 10240

    token2 = token_x.reshape(-1, d // 2)
    acc, cnt = _sc_accumulate(token2, t2r_edge_index, edge_attr_t2r, n_pad)

    blk = 1024
    grid = -(-n_rows // blk)
    row_new = pl.pallas_call(
        _tc_body,
        grid=(grid,),
        in_specs=[
            pl.BlockSpec((blk, d + de), lambda i: (i, 0)),
            pl.BlockSpec((blk, 16), lambda i: (i, 0)),
            pl.BlockSpec((blk, d), lambda i: (i, 0)),
            pl.BlockSpec((d + de, d), lambda i: (0, 0)),
            pl.BlockSpec((1, d), lambda i: (0, 0)),
            pl.BlockSpec((1, d), lambda i: (0, 0)),
            pl.BlockSpec((1, d), lambda i: (0, 0)),
        ],
        out_specs=pl.BlockSpec((blk, d), lambda i: (i, 0)),
        out_shape=jax.ShapeDtypeStruct((n_rows, d), jnp.float32),
    )(acc, cnt, row_x, W, b.reshape(1, d), gamma.reshape(1, d),
      beta.reshape(1, d))
    return (row_new, token_x)
